# Initial kernel scaffold; baseline (speedup 1.0000x reference)
import jax
import jax.numpy as jnp
from jax.experimental import pallas as pl


def kernel(tokens, table):
    raise NotImplementedError("write your pallas kernel here")


# speedup vs baseline: 2.2815x; 2.2815x over previous
import functools

import jax
import jax.numpy as jnp
from jax import lax
from jax.experimental import pallas as pl
from jax.experimental.pallas import tpu as pltpu
from jax.experimental.pallas import tpu_sc as plsc

_VOCAB = 100000
_DIM = 300
_BATCH = 16384
_TAIL = _DIM - 256

_NC = 2
_NS = 16
_NW = _NC * _NS
_CHUNK = 128
_CPW = _BATCH // (_NW * _CHUNK)
_BPW = _BATCH // _NW


def _embed_body(idx_hbm, table_hbm, tail_hbm, out_hbm,
                idx_v, bufs0, bufs1, sem0, sem1):
    wid = lax.axis_index("s") * _NC + lax.axis_index("c")
    pltpu.sync_copy(idx_hbm.at[pl.ds(wid * _BPW, _BPW)], idx_v)
    bufs = (bufs0, bufs1)
    sems = (sem0, sem1)

    def start(j):
        b = j % 2
        ii = idx_v.at[pl.ds(j * _CHUNK, _CHUNK)]
        return (
            pltpu.async_copy(table_hbm.at[ii, pl.ds(0, 128)], bufs[b][0], sems[b]),
            pltpu.async_copy(table_hbm.at[ii, pl.ds(128, 128)], bufs[b][1], sems[b]),
            pltpu.async_copy(tail_hbm.at[ii], bufs[b][2], sems[b]),
        )

    copies = [start(0), None]
    for j in range(_CPW):
        b = j % 2
        if j + 1 < _CPW:
            copies[(j + 1) % 2] = start(j + 1)
        for cp in copies[b]:
            cp.wait()
        row0 = (wid * _CPW + j) * _CHUNK
        rows = pl.ds(row0, _CHUNK)
        pltpu.sync_copy(bufs[b][0], out_hbm.at[rows, pl.ds(0, 128)])
        pltpu.sync_copy(bufs[b][1], out_hbm.at[rows, pl.ds(128, 128)])
        pltpu.sync_copy(bufs[b][2], out_hbm.at[rows, pl.ds(256, 128)])


_embed_lookup = functools.partial(
    pl.kernel,
    out_type=jax.ShapeDtypeStruct((_BATCH, 384), jnp.float32),
    mesh=plsc.VectorSubcoreMesh(core_axis_name="c", subcore_axis_name="s"),
    scratch_types=[
        pltpu.VMEM((_BPW,), jnp.int32),
        tuple(pltpu.VMEM((_CHUNK, 128), jnp.float32) for _ in range(3)),
        tuple(pltpu.VMEM((_CHUNK, 128), jnp.float32) for _ in range(3)),
        pltpu.SemaphoreType.DMA,
        pltpu.SemaphoreType.DMA,
    ],
)(_embed_body)


def kernel(tokens, table):
    idx = tokens.astype(jnp.int32)
    tail = jnp.pad(table[:, 256:], ((0, 0), (0, 128 - _TAIL)))
    out_pad = _embed_lookup(idx, table, tail)
    return out_pad[:, :_DIM]


# speedup vs baseline: 2.6258x; 1.1509x over previous
import functools

import jax
import jax.numpy as jnp
from jax import lax
from jax.experimental import pallas as pl
from jax.experimental.pallas import tpu as pltpu
from jax.experimental.pallas import tpu_sc as plsc

_VOCAB = 100000
_DIM = 300
_BATCH = 16384
_TAIL = _DIM - 256

_NC = 2
_NS = 16
_NW = _NC * _NS
_CHUNK = 128
_CPW = _BATCH // (_NW * _CHUNK)
_BPW = _BATCH // _NW


def _embed_body(idx_hbm, table_hbm, tail_hbm, out_hbm,
                idx_v, bufs0, bufs1, sem0, sem1):
    wid = lax.axis_index("s") * _NC + lax.axis_index("c")
    pltpu.sync_copy(idx_hbm.at[pl.ds(wid * _BPW, _BPW)], idx_v)
    bufs = (bufs0, bufs1)
    sems = (sem0, sem1)

    def start(j):
        b = j % 2
        ii = idx_v.at[pl.ds(j * _CHUNK, _CHUNK)]
        return (
            pltpu.async_copy(table_hbm.at[ii, pl.ds(0, 128)], bufs[b][0], sems[b]),
            pltpu.async_copy(table_hbm.at[ii, pl.ds(128, 128)], bufs[b][1], sems[b]),
            pltpu.async_copy(tail_hbm.at[ii], bufs[b][2], sems[b]),
        )

    copies = [start(0), None]
    for j in range(_CPW):
        b = j % 2
        if j + 1 < _CPW:
            copies[(j + 1) % 2] = start(j + 1)
        for cp in copies[b]:
            cp.wait()
        row0 = (wid * _CPW + j) * _CHUNK
        rows = pl.ds(row0, _CHUNK)
        pltpu.sync_copy(bufs[b][0], out_hbm.at[rows, pl.ds(0, 128)])
        pltpu.sync_copy(bufs[b][1], out_hbm.at[rows, pl.ds(128, 128)])
        pltpu.sync_copy(bufs[b][2], out_hbm.at[rows, pl.ds(256, 128)])


_embed_lookup = functools.partial(
    pl.kernel,
    out_type=jax.ShapeDtypeStruct((_BATCH, 384), jnp.float32),
    mesh=plsc.VectorSubcoreMesh(core_axis_name="c", subcore_axis_name="s"),
    scratch_types=[
        pltpu.VMEM((_BPW,), jnp.int32),
        tuple(pltpu.VMEM((_CHUNK, 128), jnp.float32) for _ in range(3)),
        tuple(pltpu.VMEM((_CHUNK, 128), jnp.float32) for _ in range(3)),
        pltpu.SemaphoreType.DMA,
        pltpu.SemaphoreType.DMA,
    ],
)(_embed_body)


def kernel(tokens, table):
    idx = tokens.astype(jnp.int32)
    tail = lax.pad(table, jnp.float32(0), [(0, 0, 0), (-256, 128 - _TAIL, 0)])
    out_pad = _embed_lookup(idx, table, tail)
    return out_pad[:, :_DIM]
